# baseline (speedup 1.0000x reference)
import functools

import jax
import jax.numpy as jnp
from jax import lax
from jax.experimental import pallas as pl
from jax.experimental.pallas import tpu as pltpu
from jax.experimental.pallas import tpu_sc as plsc

N_ROWS = 1_000_000
EMBED_DIM = 32
BATCH = 16384
HIST = 50

NUM_CORES = 2
NUM_SUBCORES = 16
NUM_WORKERS = NUM_CORES * NUM_SUBCORES
LANES = 16

B_PER_W = BATCH // NUM_WORKERS
SUB = 16


def _make_gather():
    mesh = plsc.VectorSubcoreMesh(core_axis_name="c", subcore_axis_name="s")

    @functools.partial(
        pl.kernel,
        mesh=mesh,
        out_type=jax.ShapeDtypeStruct((HIST, EMBED_DIM, BATCH), jnp.float32),
        scratch_types=[
            pltpu.VMEM((B_PER_W, HIST), jnp.int32),
            pltpu.VMEM((HIST, B_PER_W), jnp.int32),
            pltpu.VMEM((B_PER_W, EMBED_DIM), jnp.float32),
            pltpu.VMEM((B_PER_W, EMBED_DIM), jnp.float32),
            pltpu.VMEM((EMBED_DIM, B_PER_W + 1), jnp.float32),
            pltpu.SemaphoreType.DMA,
            pltpu.SemaphoreType.DMA,
        ],
        compiler_params=pltpu.CompilerParams(
            use_tc_tiling_on_sc=False, needs_layout_passes=False,
            disable_bounds_checks=True),
    )
    def gather_kernel(x_hbm, table_hbm, out_hbm, idx_v, idxt_v, rows0, rows1,
                      trans_v, g0, g1):
        wid = lax.axis_index("s") * NUM_CORES + lax.axis_index("c")
        b0 = wid * B_PER_W
        rows = (rows0, rows1)
        gsem = (g0, g1)
        lane = lax.broadcasted_iota(jnp.int32, (LANES,), 0)

        pltpu.sync_copy(x_hbm.at[pl.ds(b0, B_PER_W), :], idx_v)

        def it_body(j, carry):
            col = jnp.full((LANES,), j, jnp.int32)
            for l0 in range(0, B_PER_W, LANES):
                v = plsc.load_gather(idx_v, [lane + l0, col])
                idxt_v[j, pl.ds(l0, LANES)] = v
            return carry

        lax.fori_loop(0, HIST, it_body, 0, unroll=False)

        def gstart(j, b):
            pltpu.async_copy(table_hbm.at[idxt_v.at[j]], rows[b], gsem[b])

        def gwait(j, b):
            pltpu.make_async_copy(table_hbm.at[idxt_v.at[j]], rows[b],
                                  gsem[b]).wait()

        def transpose_and_store(j, b):
            def b_body(bb, carry):
                for s in range(SUB):
                    brow = bb * SUB + s
                    col = jnp.full((LANES,), brow, jnp.int32)
                    for c0 in (0, LANES):
                        v = rows[b][brow, pl.ds(c0, LANES)]
                        plsc.store_scatter(trans_v, [lane + c0, col], v)
                return carry

            lax.fori_loop(0, B_PER_W // SUB, b_body, 0, unroll=False)
            pltpu.sync_copy(
                trans_v.at[:, pl.ds(0, B_PER_W)],
                out_hbm.at[j, :, pl.ds(b0, B_PER_W)])

        gstart(0, 0)

        def pair_body(j2, carry):
            j = 2 * j2
            gwait(j, 0)
            gstart(j + 1, 1)
            transpose_and_store(j, 0)
            gwait(j + 1, 1)
            gstart(j + 2, 0)
            transpose_and_store(j + 1, 1)
            return carry

        lax.fori_loop(0, HIST // 2 - 1, pair_body, 0, unroll=False)
        gwait(HIST - 2, 0)
        gstart(HIST - 1, 1)
        transpose_and_store(HIST - 2, 0)
        gwait(HIST - 1, 1)
        transpose_and_store(HIST - 1, 1)

    return gather_kernel


_gather = _make_gather()


def kernel(x, W):
    out_t = _gather(x.astype(jnp.int32), W)
    return lax.transpose(out_t, (2, 0, 1))


# speedup vs baseline: 1.0000x; 1.0000x over previous
import functools

import jax
import jax.numpy as jnp
from jax import lax
from jax.experimental import pallas as pl
from jax.experimental.pallas import tpu as pltpu
from jax.experimental.pallas import tpu_sc as plsc

N_ROWS = 1_000_000
EMBED_DIM = 32
BATCH = 16384
HIST = 50

NUM_CORES = 2
NUM_SUBCORES = 16
NUM_WORKERS = NUM_CORES * NUM_SUBCORES
LANES = 16

B_PER_W = BATCH // NUM_WORKERS
SUB = 16


def _make_gather():
    mesh = plsc.VectorSubcoreMesh(core_axis_name="c", subcore_axis_name="s")

    @functools.partial(
        pl.kernel,
        mesh=mesh,
        out_type=jax.ShapeDtypeStruct((HIST, EMBED_DIM, BATCH), jnp.float32),
        scratch_types=[
            pltpu.VMEM((B_PER_W, HIST), jnp.float32),
            pltpu.VMEM((HIST, B_PER_W), jnp.int32),
            pltpu.VMEM((B_PER_W, EMBED_DIM), jnp.float32),
            pltpu.VMEM((B_PER_W, EMBED_DIM), jnp.float32),
            pltpu.VMEM((EMBED_DIM, B_PER_W + 1), jnp.float32),
            pltpu.SemaphoreType.DMA,
            pltpu.SemaphoreType.DMA,
        ],
        compiler_params=pltpu.CompilerParams(
            use_tc_tiling_on_sc=False, needs_layout_passes=False,
            disable_bounds_checks=True),
    )
    def gather_kernel(x_hbm, table_hbm, out_hbm, idx_v, idxt_v, rows0, rows1,
                      trans_v, g0, g1):
        wid = lax.axis_index("s") * NUM_CORES + lax.axis_index("c")
        b0 = wid * B_PER_W
        rows = (rows0, rows1)
        gsem = (g0, g1)
        lane = lax.broadcasted_iota(jnp.int32, (LANES,), 0)

        pltpu.sync_copy(x_hbm.at[pl.ds(b0, B_PER_W), :], idx_v)

        def it_body(j, carry):
            col = jnp.full((LANES,), j, jnp.int32)
            for l0 in range(0, B_PER_W, LANES):
                v = plsc.load_gather(idx_v, [lane + l0, col])
                idxt_v[j, pl.ds(l0, LANES)] = plsc.bitcast(v, jnp.int32)
            return carry

        lax.fori_loop(0, HIST, it_body, 0, unroll=False)

        def gstart(j, b):
            pltpu.async_copy(table_hbm.at[idxt_v.at[j]], rows[b], gsem[b])

        def gwait(j, b):
            pltpu.make_async_copy(table_hbm.at[idxt_v.at[j]], rows[b],
                                  gsem[b]).wait()

        def transpose_and_store(j, b):
            def b_body(bb, carry):
                for s in range(SUB):
                    brow = bb * SUB + s
                    col = jnp.full((LANES,), brow, jnp.int32)
                    for c0 in (0, LANES):
                        v = rows[b][brow, pl.ds(c0, LANES)]
                        plsc.store_scatter(trans_v, [lane + c0, col], v)
                return carry

            lax.fori_loop(0, B_PER_W // SUB, b_body, 0, unroll=False)
            pltpu.sync_copy(
                trans_v.at[:, pl.ds(0, B_PER_W)],
                out_hbm.at[j, :, pl.ds(b0, B_PER_W)])

        gstart(0, 0)

        def pair_body(j2, carry):
            j = 2 * j2
            gwait(j, 0)
            gstart(j + 1, 1)
            transpose_and_store(j, 0)
            gwait(j + 1, 1)
            gstart(j + 2, 0)
            transpose_and_store(j + 1, 1)
            return carry

        lax.fori_loop(0, HIST // 2 - 1, pair_body, 0, unroll=False)
        gwait(HIST - 2, 0)
        gstart(HIST - 1, 1)
        transpose_and_store(HIST - 2, 0)
        gwait(HIST - 1, 1)
        transpose_and_store(HIST - 1, 1)

    return gather_kernel


_gather = _make_gather()


def kernel(x, W):
    x_bits = lax.bitcast_convert_type(x.astype(jnp.int32), jnp.float32)
    out_t = _gather(x_bits, W)
    return lax.transpose(out_t, (2, 0, 1))


# speedup vs baseline: 1.0288x; 1.0288x over previous
import functools

import jax
import jax.numpy as jnp
from jax import lax
from jax.experimental import pallas as pl
from jax.experimental.pallas import tpu as pltpu
from jax.experimental.pallas import tpu_sc as plsc

N_ROWS = 1_000_000
EMBED_DIM = 32
BATCH = 16384
HIST = 50

NUM_CORES = 2
NUM_SUBCORES = 16
NUM_WORKERS = NUM_CORES * NUM_SUBCORES
LANES = 16

B_PER_W = BATCH // NUM_WORKERS
SUB = 16


def _make_gather():
    mesh = plsc.VectorSubcoreMesh(core_axis_name="c", subcore_axis_name="s")

    @functools.partial(
        pl.kernel,
        mesh=mesh,
        out_type=jax.ShapeDtypeStruct((HIST, EMBED_DIM, BATCH), jnp.float32),
        scratch_types=[
            pltpu.VMEM((B_PER_W, HIST), jnp.float32),
            pltpu.VMEM((HIST, B_PER_W), jnp.int32),
            pltpu.VMEM((B_PER_W, EMBED_DIM), jnp.float32),
            pltpu.VMEM((B_PER_W, EMBED_DIM), jnp.float32),
            pltpu.VMEM((EMBED_DIM, B_PER_W + 1), jnp.float32),
            pltpu.VMEM((EMBED_DIM, B_PER_W + 1), jnp.float32),
            pltpu.SemaphoreType.DMA,
            pltpu.SemaphoreType.DMA,
            pltpu.SemaphoreType.DMA,
            pltpu.SemaphoreType.DMA,
        ],
        compiler_params=pltpu.CompilerParams(
            use_tc_tiling_on_sc=False, needs_layout_passes=False,
            disable_bounds_checks=True),
    )
    def gather_kernel(x_hbm, table_hbm, out_hbm, idx_v, idxt_v, rows0, rows1,
                      trans0, trans1, g0, g1, w0, w1):
        wid = lax.axis_index("s") * NUM_CORES + lax.axis_index("c")
        b0 = wid * B_PER_W
        rows = (rows0, rows1)
        trans = (trans0, trans1)
        gsem = (g0, g1)
        wsem = (w0, w1)
        lane = lax.broadcasted_iota(jnp.int32, (LANES,), 0)

        pltpu.sync_copy(x_hbm.at[pl.ds(b0, B_PER_W), :], idx_v)

        def it_body(j, carry):
            col = jnp.full((LANES,), j, jnp.int32)
            for l0 in range(0, B_PER_W, LANES):
                v = plsc.load_gather(idx_v, [lane + l0, col])
                idxt_v[j, pl.ds(l0, LANES)] = plsc.bitcast(v, jnp.int32)
            return carry

        lax.fori_loop(0, HIST, it_body, 0, unroll=False)

        def gstart(j, b):
            pltpu.async_copy(table_hbm.at[idxt_v.at[j]], rows[b], gsem[b])

        def gwait(j, b):
            pltpu.make_async_copy(table_hbm.at[idxt_v.at[j]], rows[b],
                                  gsem[b]).wait()

        def transpose(j, b):
            def b_body(bb, carry):
                for s in range(SUB):
                    brow = bb * SUB + s
                    col = jnp.full((LANES,), brow, jnp.int32)
                    for c0 in (0, LANES):
                        v = rows[b][brow, pl.ds(c0, LANES)]
                        plsc.store_scatter(trans[b], [lane + c0, col], v)
                return carry

            lax.fori_loop(0, B_PER_W // SUB, b_body, 0, unroll=False)

        def wstart(j, b):
            pltpu.async_copy(
                trans[b].at[:, pl.ds(0, B_PER_W)],
                out_hbm.at[j, :, pl.ds(b0, B_PER_W)], wsem[b])

        def wwait(j, b):
            pltpu.make_async_copy(
                trans[b].at[:, pl.ds(0, B_PER_W)],
                out_hbm.at[j, :, pl.ds(b0, B_PER_W)], wsem[b]).wait()

        def step(j, b, nxt, prev):
            gwait(j, b)
            if nxt is not None:
                gstart(nxt, 1 - b)
            if prev is not None:
                wwait(prev, b)
            transpose(j, b)
            wstart(j, b)

        gstart(0, 0)
        step(0, 0, 1, None)
        step(1, 1, 2, None)

        def pair_body(j2, carry):
            j = 2 * j2
            step(j, 0, j + 1, j - 2)
            step(j + 1, 1, j + 2, j - 1)
            return carry

        lax.fori_loop(1, HIST // 2 - 1, pair_body, 0, unroll=False)
        step(HIST - 2, 0, HIST - 1, HIST - 4)
        step(HIST - 1, 1, None, HIST - 3)
        wwait(HIST - 2, 0)
        wwait(HIST - 1, 1)

    return gather_kernel


_gather = _make_gather()


def kernel(x, W):
    x_bits = lax.bitcast_convert_type(x.astype(jnp.int32), jnp.float32)
    out_t = _gather(x_bits, W)
    return lax.transpose(out_t, (2, 0, 1))
